# Initial kernel scaffold; baseline (speedup 1.0000x reference)
import jax
import jax.numpy as jnp
from jax.experimental import pallas as pl


def kernel(x):
    raise NotImplementedError("write your pallas kernel here")


# speedup vs baseline: 15.4804x; 15.4804x over previous
import functools

import jax
import jax.numpy as jnp
from jax import lax
from jax.experimental import pallas as pl
from jax.experimental.pallas import tpu as pltpu
from jax.experimental.pallas import tpu_sc as plsc

B = 16384
D = 16
K = 64
OUTW = D * (K + 1)
SCALE = (K - 3) / 2.0
CLIP_HI = K - 3 - 1e-06
MIN_VAL = -1.0

NW = 32
ROWS_PW = B // NW
CHUNK = 32
NCHUNK = ROWS_PW // CHUNK
L = 16


def _body(x_hbm, out_hbm, xv, ob0, ob1, sem0, sem1):
    wid = lax.axis_index("s") * 2 + lax.axis_index("c")
    wbase = wid * ROWS_PW

    pltpu.sync_copy(x_hbm.at[pl.ds(wbase, ROWS_PW)], xv)

    dcol = lax.iota(jnp.int32, L) * (K + 1)
    zero16 = jnp.zeros((L,), jnp.float32)
    obufs = (ob0, ob1)
    sems = (sem0, sem1)

    for k in range(NCHUNK):
        slot = k & 1
        ob = obufs[slot]
        sem = sems[slot]
        if k >= 2:
            pltpu.make_async_copy(
                obufs[slot], out_hbm.at[pl.ds(wbase + (k - 2) * CHUNK, CHUNK)], sem
            ).wait()

        @pl.loop(0, CHUNK)
        def _zero_row(r):
            @pl.loop(0, OUTW // L)
            def _zero_seg(j):
                ob[r, pl.ds(j * L, L)] = zero16

        @pl.loop(0, CHUNK)
        def _row(r):
            x16 = xv[k * CHUNK + r]
            t = (x16 - MIN_VAL) * SCALE
            t = jnp.minimum(jnp.maximum(t, 0.0), CLIP_HI)
            idx = t.astype(jnp.int32)
            u = t - idx.astype(jnp.float32)
            u2 = u * u
            u3 = u2 * u
            w = 1.0 - u
            c0 = w * w * w / 6.0
            c1 = (3.0 * u3 - 6.0 * u2 + 4.0) / 6.0
            c2 = (-3.0 * u3 + 3.0 * u2 + 3.0 * u + 1.0) / 6.0
            c3 = u3 / 6.0
            rvec = jnp.broadcast_to(r, (L,)).astype(jnp.int32)
            plsc.store_scatter(ob, [rvec, dcol], x16)
            cb = dcol + 1 + idx
            plsc.store_scatter(ob, [rvec, cb], c0)
            plsc.store_scatter(ob, [rvec, cb + 1], c1)
            plsc.store_scatter(ob, [rvec, cb + 2], c2)
            plsc.store_scatter(ob, [rvec, cb + 3], c3)

        pltpu.async_copy(ob, out_hbm.at[pl.ds(wbase + k * CHUNK, CHUNK)], sem)

    for k in (NCHUNK - 2, NCHUNK - 1):
        slot = k & 1
        pltpu.make_async_copy(
            obufs[slot], out_hbm.at[pl.ds(wbase + k * CHUNK, CHUNK)], sems[slot]
        ).wait()


@jax.jit
def _encode(x):
    mesh = plsc.VectorSubcoreMesh(
        core_axis_name="c", subcore_axis_name="s", num_cores=2, num_subcores=16
    )
    run = pl.kernel(
        _body,
        out_type=jax.ShapeDtypeStruct((B, OUTW), jnp.float32),
        mesh=mesh,
        compiler_params=pltpu.CompilerParams(
            use_tc_tiling_on_sc=False, needs_layout_passes=False
        ),
        scratch_types=[
            pltpu.VMEM((ROWS_PW, D), jnp.float32),
            pltpu.VMEM((CHUNK, OUTW), jnp.float32),
            pltpu.VMEM((CHUNK, OUTW), jnp.float32),
            pltpu.SemaphoreType.DMA,
            pltpu.SemaphoreType.DMA,
        ],
    )
    return run(x)


def kernel(x):
    return _encode(x)


# speedup vs baseline: 26.0738x; 1.6843x over previous
import jax
import jax.numpy as jnp
from jax import lax
from jax.experimental import pallas as pl
from jax.experimental.pallas import tpu as pltpu
from jax.experimental.pallas import tpu_sc as plsc

B = 16384
D = 16
K = 64
OUTW = D * (K + 1)
SCALE = (K - 3) / 2.0
CLIP_HI = K - 3 - 1e-06
MIN_VAL = -1.0

NW = 32
ROWS_PW = B // NW
CHUNK = 32
NCHUNK = ROWS_PW // CHUNK
L = 16


def _colbase(x16, dcol):
    t = (x16 - MIN_VAL) * SCALE
    t = jnp.minimum(jnp.maximum(t, 0.0), CLIP_HI)
    idx = t.astype(jnp.int32)
    u = t - idx.astype(jnp.float32)
    return dcol + 1 + idx, u


def _body(x_hbm, z_hbm, out_hbm, xv, ob0, ob1, semz, sem0, sem1):
    wid = lax.axis_index("s") * 2 + lax.axis_index("c")
    wbase = wid * ROWS_PW

    dcol = lax.iota(jnp.int32, L) * (K + 1)
    zero16 = jnp.zeros((L,), jnp.float32)
    obufs = (ob0, ob1)
    sems = (sem0, sem1)

    pltpu.async_copy(z_hbm, ob0, semz)
    pltpu.async_copy(z_hbm, ob1, semz)
    pltpu.sync_copy(x_hbm.at[pl.ds(wbase, ROWS_PW)], xv)
    pltpu.make_async_copy(z_hbm, ob0, semz).wait()
    pltpu.make_async_copy(z_hbm, ob1, semz).wait()

    def emit_chunk(k, slot, restore):
        ob = obufs[slot]

        @pl.loop(0, CHUNK, unroll=4)
        def _row(r):
            rvec = jnp.broadcast_to(r, (L,)).astype(jnp.int32)
            if restore:
                cbo, _ = _colbase(xv[(k - 2) * CHUNK + r], dcol)
                plsc.store_scatter(ob, [rvec, cbo], zero16)
                plsc.store_scatter(ob, [rvec, cbo + 1], zero16)
                plsc.store_scatter(ob, [rvec, cbo + 2], zero16)
                plsc.store_scatter(ob, [rvec, cbo + 3], zero16)
            x16 = xv[k * CHUNK + r]
            cb, u = _colbase(x16, dcol)
            u2 = u * u
            u3 = u2 * u
            w = 1.0 - u
            c0 = w * w * w / 6.0
            c1 = (3.0 * u3 - 6.0 * u2 + 4.0) / 6.0
            c2 = (-3.0 * u3 + 3.0 * u2 + 3.0 * u + 1.0) / 6.0
            c3 = u3 / 6.0
            plsc.store_scatter(ob, [rvec, dcol], x16)
            plsc.store_scatter(ob, [rvec, cb], c0)
            plsc.store_scatter(ob, [rvec, cb + 1], c1)
            plsc.store_scatter(ob, [rvec, cb + 2], c2)
            plsc.store_scatter(ob, [rvec, cb + 3], c3)

        pltpu.async_copy(ob, out_hbm.at[pl.ds(wbase + k * CHUNK, CHUNK)], sems[slot])

    emit_chunk(0, 0, restore=False)
    emit_chunk(1, 1, restore=False)

    @pl.loop(2, NCHUNK, step=2)
    def _chunks(k0):
        for b in range(2):
            k = k0 + b
            pltpu.make_async_copy(
                obufs[b], out_hbm.at[pl.ds(wbase + (k - 2) * CHUNK, CHUNK)], sems[b]
            ).wait()
            emit_chunk(k, b, restore=True)

    for k in (NCHUNK - 2, NCHUNK - 1):
        slot = k & 1
        pltpu.make_async_copy(
            obufs[slot], out_hbm.at[pl.ds(wbase + k * CHUNK, CHUNK)], sems[slot]
        ).wait()


@jax.jit
def _encode(x):
    mesh = plsc.VectorSubcoreMesh(
        core_axis_name="c", subcore_axis_name="s", num_cores=2, num_subcores=16
    )
    run = pl.kernel(
        _body,
        out_type=jax.ShapeDtypeStruct((B, OUTW), jnp.float32),
        mesh=mesh,
        compiler_params=pltpu.CompilerParams(
            use_tc_tiling_on_sc=False, needs_layout_passes=False
        ),
        scratch_types=[
            pltpu.VMEM((ROWS_PW, D), jnp.float32),
            pltpu.VMEM((CHUNK, OUTW), jnp.float32),
            pltpu.VMEM((CHUNK, OUTW), jnp.float32),
            pltpu.SemaphoreType.DMA,
            pltpu.SemaphoreType.DMA,
            pltpu.SemaphoreType.DMA,
        ],
    )
    zeros = jnp.zeros((CHUNK, OUTW), jnp.float32)
    return run(x, zeros)


def kernel(x):
    return _encode(x)


# speedup vs baseline: 91.7215x; 3.5178x over previous
import jax
import jax.numpy as jnp
from jax import lax
from jax.experimental import pallas as pl
from jax.experimental.pallas import tpu as pltpu
from jax.experimental.pallas import tpu_sc as plsc

B = 16384
D = 16
K = 64
OUTW = D * (K + 1)
SCALE = (K - 3) / 2.0
CLIP_HI = K - 3 - 1e-06
MIN_VAL = -1.0

C8 = OUTW // 8
NB128 = B // 128
NW = 32
BPW = NB128 // NW
L = 16

F0S = (0, 256, 520, 776)
QCFG = (
    dict(fs=0, fd=0, ps=32, pr=0, pd=32, pdr=0),
    dict(fs=1, fd=33, ps=0, pr=4, pd=32, pdr=4),
    dict(fs=0, fd=65, ps=32, pr=0, pd=97, pdr=0),
    dict(fs=1, fd=98, ps=0, pr=4, pd=97, pdr=4),
)


def _scaled(xv):
    t = (xv - MIN_VAL) * SCALE
    t = jnp.minimum(jnp.maximum(t, 0.0), CLIP_HI)
    idx = t.astype(jnp.int32)
    return idx, t


def _copies(out_hbm, buf, sem, q, b128):
    c = QCFG[q]
    return (
        pltpu.make_async_copy(
            buf.at[pl.ds(c["fs"], 32)], out_hbm.at[pl.ds(c["fd"], 32), b128], sem
        ),
        pltpu.make_async_copy(
            buf.at[pl.ds(c["ps"], 1), pl.ds(c["pr"], 4)],
            out_hbm.at[pl.ds(c["pd"], 1), b128, pl.ds(c["pdr"], 4)],
            sem,
        ),
    )


def _body(x4_hbm, z_hbm, out_hbm, xb, buf0, buf1, sem0, sem1):
    wid = lax.axis_index("s") * 2 + lax.axis_index("c")
    bbase = wid * BPW

    bufs = (buf0, buf1)
    sems = (sem0, sem1)
    zero16 = jnp.zeros((L,), jnp.float32)

    pltpu.async_copy(z_hbm, buf0, sem0)
    pltpu.async_copy(z_hbm, buf1, sem1)
    pltpu.sync_copy(x4_hbm.at[:, pl.ds(bbase, BPW)], xb)
    pltpu.make_async_copy(z_hbm, buf0, sem0).wait()
    pltpu.make_async_copy(z_hbm, buf1, sem1).wait()

    def emit(q, jj, restore, q_old, jj_old):
        buf = bufs[q & 1]
        F0 = F0S[q]

        @pl.loop(0, 8)
        def _i(i):
            off = i * L
            lanes = lax.iota(jnp.int32, L) + off
            if restore:
                F0o = F0S[q_old]
                for dd in range(4):
                    d_o = 4 * q_old + dd
                    xo = xb[d_o >> 3, jj_old, d_o & 7, pl.ds(off, L)]
                    idxo, _ = _scaled(xo)
                    r0 = idxo + (d_o * 65 + 1 - F0o)
                    for jc in range(4):
                        row = r0 + jc
                        plsc.store_scatter(buf, [row >> 3, row & 7, lanes], zero16)
            for dd in range(4):
                d = 4 * q + dd
                xv = xb[d >> 3, jj, d & 7, pl.ds(off, L)]
                idx, t = _scaled(xv)
                u = t - idx.astype(jnp.float32)
                u2 = u * u
                u3 = u2 * u
                w = 1.0 - u
                c0 = w * w * w / 6.0
                c1 = (3.0 * u3 - 6.0 * u2 + 4.0) / 6.0
                c2 = (-3.0 * u3 + 3.0 * u2 + 3.0 * u + 1.0) / 6.0
                c3 = u3 / 6.0
                rx = d * 65 - F0
                buf[rx >> 3, rx & 7, pl.ds(off, L)] = xv
                r0 = idx + (d * 65 + 1 - F0)
                for jc, cc in enumerate((c0, c1, c2, c3)):
                    row = r0 + jc
                    plsc.store_scatter(buf, [row >> 3, row & 7, lanes], cc)

    def start(q, b128):
        for cp in _copies(out_hbm, bufs[q & 1], sems[q & 1], q, b128):
            cp.start()

    def drain(q, b128):
        for cp in _copies(out_hbm, bufs[q & 1], sems[q & 1], q, b128):
            cp.wait()

    emit(0, 0, False, 0, 0)
    start(0, bbase)
    emit(1, 0, False, 0, 0)
    start(1, bbase)
    drain(0, bbase)
    emit(2, 0, True, 0, 0)
    start(2, bbase)
    drain(1, bbase)
    emit(3, 0, True, 1, 0)
    start(3, bbase)

    @pl.loop(1, BPW)
    def _j(jj):
        b128 = bbase + jj
        drain(2, b128 - 1)
        emit(0, jj, True, 2, jj - 1)
        start(0, b128)
        drain(3, b128 - 1)
        emit(1, jj, True, 3, jj - 1)
        start(1, b128)
        drain(0, b128)
        emit(2, jj, True, 0, jj)
        start(2, b128)
        drain(1, b128)
        emit(3, jj, True, 1, jj)
        start(3, b128)

    drain(2, bbase + BPW - 1)
    drain(3, bbase + BPW - 1)


@jax.jit
def _encode(x):
    mesh = plsc.VectorSubcoreMesh(
        core_axis_name="c", subcore_axis_name="s", num_cores=2, num_subcores=16
    )
    run = pl.kernel(
        _body,
        out_type=jax.ShapeDtypeStruct((C8, NB128, 8, 128), jnp.float32),
        mesh=mesh,
        compiler_params=pltpu.CompilerParams(
            use_tc_tiling_on_sc=False, needs_layout_passes=False
        ),
        scratch_types=[
            pltpu.VMEM((2, BPW, 8, 128), jnp.float32),
            pltpu.VMEM((33, 8, 128), jnp.float32),
            pltpu.VMEM((33, 8, 128), jnp.float32),
            pltpu.SemaphoreType.DMA,
            pltpu.SemaphoreType.DMA,
        ],
    )
    x4 = jnp.transpose(x.T.reshape(2, 8, NB128, 128), (0, 2, 1, 3))
    zeros = jnp.zeros((33, 8, 128), jnp.float32)
    out4 = run(x4, zeros)
    return jnp.transpose(out4, (1, 3, 0, 2)).reshape(B, OUTW)


def kernel(x):
    return _encode(x)
